# baseline (speedup 1.0000x reference)
import functools

import jax
import jax.numpy as jnp
from jax import lax
from jax.experimental import pallas as pl
from jax.experimental.pallas import tpu as pltpu
from jax.experimental.pallas import tpu_sc as plsc

N = 10000
D = 128
E = 320000
HC = D // 2
U = 128
NU = E // U
UB = 8

_NC = 2
_NS = 16
_NW = _NC * _NS
NU_W = -(-NU // _NW)
CH = 40
NCH = N // CH

_SC_PARAMS = pltpu.CompilerParams(use_tc_tiling_on_sc=False)


def _zero_vmem(buf, rows, cols):
    def zrow(r, carry):
        for k in range(cols // 16):
            buf[r, pl.ds(k * 16, 16)] = jnp.zeros((16,), jnp.float32)
        return carry

    lax.fori_loop(0, rows, zrow, 0)


@functools.partial(
    pl.kernel,
    out_type=(
        jax.ShapeDtypeStruct((N, HC), jnp.float32),
        jax.ShapeDtypeStruct((N, HC), jnp.float32),
    ),
    mesh=plsc.VectorSubcoreMesh(core_axis_name="c", subcore_axis_name="s"),
    scratch_types=[
        pltpu.VMEM_SHARED((N, HC), jnp.float32),
        pltpu.VMEM((CH, HC), jnp.float32),
        pltpu.VMEM((UB, U), jnp.int32),
        pltpu.VMEM((UB, U), jnp.int32),
    ] + [pltpu.VMEM((U, HC), jnp.float32)] * UB + [
        pltpu.SemaphoreType.DMA((UB,)),
        pltpu.SemaphoreType.DMA((UB,)),
    ],
    compiler_params=_SC_PARAMS,
)
def _sc_scatter(xcat, srccat, dst2d, aggA, aggB,
                shared_agg, zbuf, src_b, dst_b, *rest):
    rows = rest[:UB]
    gsem, ssem = rest[UB], rest[UB + 1]
    c = lax.axis_index("c")
    s = lax.axis_index("s")

    _zero_vmem(zbuf, CH, HC)

    def ztile(t, carry):
        pltpu.sync_copy(zbuf, shared_agg.at[pl.ds(t * CH, CH)])
        return carry

    lax.fori_loop(NCH * s // _NS, NCH * (s + 1) // _NS, ztile, 0)
    plsc.subcore_barrier()

    u0 = NU * s // _NS
    u1 = NU * (s + 1) // _NS
    nbt = (u1 - u0 + UB - 1) // UB

    def wait_gather(j):
        pltpu.make_async_copy(xcat.at[pl.ds(0, U)], rows[j], gsem.at[j]).wait()

    def wait_scatter(j):
        pltpu.make_async_copy(rows[j], shared_agg.at[pl.ds(0, U)],
                              ssem.at[j]).wait()

    def batch(kb, carry):
        base = u0 + kb * UB

        @pl.when(kb > 0)
        def _():
            for j in range(UB):
                wait_scatter(j)

        pltpu.sync_copy(srccat.at[c, pl.ds(base, UB)], src_b)
        pltpu.sync_copy(dst2d.at[pl.ds(base, UB)], dst_b)

        for j in range(UB):
            @pl.when(base + j < u1)
            def _(j=j):
                pltpu.async_copy(xcat.at[src_b.at[j]], rows[j], gsem.at[j])

        for j in range(UB):
            @pl.when(base + j < u1)
            def _(j=j):
                wait_gather(j)
                pltpu.async_copy(rows[j], shared_agg.at[dst_b.at[j]],
                                 ssem.at[j], add=True)

        return carry

    lax.fori_loop(0, nbt, batch, 0)

    last_base = u0 + (nbt - 1) * UB
    for j in range(UB):
        @pl.when(last_base + j < u1)
        def _(j=j):
            wait_scatter(j)

    plsc.subcore_barrier()

    def wtile(t, carry):
        tile = pl.ds(t * CH, CH)

        @pl.when(c == 0)
        def _():
            pltpu.sync_copy(shared_agg.at[tile], aggA.at[tile])

        @pl.when(c == 1)
        def _():
            pltpu.sync_copy(shared_agg.at[tile], aggB.at[tile])

        return carry

    lax.fori_loop(NCH * s // _NS, NCH * (s + 1) // _NS, wtile, 0)


@functools.partial(
    pl.kernel,
    out_type=(
        jax.ShapeDtypeStruct((N, 16), jnp.float32),
        jax.ShapeDtypeStruct((N, 16), jnp.float32),
    ),
    mesh=plsc.VectorSubcoreMesh(core_axis_name="c", subcore_axis_name="s"),
    scratch_types=[
        pltpu.VMEM_SHARED((N, 16), jnp.float32),
        pltpu.VMEM((CH, 16), jnp.float32),
        pltpu.VMEM((U, 16), jnp.float32),
        pltpu.VMEM((NU_W, U), jnp.int32),
        pltpu.SemaphoreType.DMA,
    ],
    compiler_params=_SC_PARAMS,
)
def _sc_count(dst2d, cntA, cntB, shared_cnt, zbufc, ones_v, dst_all, csem):
    c = lax.axis_index("c")
    s = lax.axis_index("s")

    _zero_vmem(zbufc, CH, 16)

    def orow(r, carry):
        ones_v[r, pl.ds(0, 16)] = jnp.ones((16,), jnp.float32)
        return carry

    lax.fori_loop(0, U, orow, 0)

    def ztile(t, carry):
        pltpu.sync_copy(zbufc, shared_cnt.at[pl.ds(t * CH, CH)])
        return carry

    lax.fori_loop(NCH * s // _NS, NCH * (s + 1) // _NS, ztile, 0)
    plsc.subcore_barrier()

    w = s * _NC + c
    u0 = NU * w // _NW
    u1 = NU * (w + 1) // _NW
    pltpu.sync_copy(dst2d.at[pl.ds(u0, NU_W)], dst_all)

    def batch(k, carry):
        base = u0 + k * 8
        for b in range(8):
            @pl.when(base + b < u1)
            def _(u=base + b):
                pltpu.async_copy(ones_v, shared_cnt.at[dst_all.at[u - u0]],
                                 csem, add=True)
        for b in range(8):
            @pl.when(base + b < u1)
            def _():
                pltpu.make_async_copy(ones_v, shared_cnt.at[pl.ds(0, U)],
                                      csem).wait()
        return carry

    lax.fori_loop(0, (u1 - u0 + 7) // 8, batch, 0)
    plsc.subcore_barrier()

    def wtile(t, carry):
        tile = pl.ds(t * CH, CH)

        @pl.when(c == 0)
        def _():
            pltpu.sync_copy(shared_cnt.at[tile], cntA.at[tile])

        @pl.when(c == 1)
        def _():
            pltpu.sync_copy(shared_cnt.at[tile], cntB.at[tile])

        return carry

    lax.fori_loop(NCH * s // _NS, NCH * (s + 1) // _NS, wtile, 0)


_B = 512


def _dense_body(aA, aB, cA, cB, x, WlT, WrT, b, out):
    cnt = cA[:, 0:1] + cB[:, 0:1]
    inv = 1.0 / jnp.maximum(cnt, 1.0)
    m = jnp.concatenate([aA[...], aB[...]], axis=1) * inv
    z = (jnp.dot(m, WlT[...], preferred_element_type=jnp.float32)
         + jnp.dot(x[...], WrT[...], preferred_element_type=jnp.float32)
         + b[...])
    out[...] = jnp.maximum(z, 0.0)


def _dense_layer(aA, aB, cA, cB, x, WlT, WrT, b):
    grid = (pl.cdiv(N, _B),)
    row128 = pl.BlockSpec((_B, D), lambda i: (i, 0))
    row64 = pl.BlockSpec((_B, HC), lambda i: (i, 0))
    row16 = pl.BlockSpec((_B, 16), lambda i: (i, 0))
    wfull = pl.BlockSpec((D, D), lambda i: (0, 0))
    bfull = pl.BlockSpec((1, D), lambda i: (0, 0))
    return pl.pallas_call(
        _dense_body,
        grid=grid,
        in_specs=[row64, row64, row16, row16, row128, wfull, wfull, bfull],
        out_specs=row128,
        out_shape=jax.ShapeDtypeStruct((N, D), jnp.float32),
    )(aA, aB, cA, cB, x, WlT, WrT, b)


def kernel(x, edge_index, W1l, b1l, W1r, W2l, b2l, W2r):
    src2d = jnp.pad(edge_index[0].astype(jnp.int32).reshape(NU, U),
                    ((0, UB), (0, 0)))
    dst2d = jnp.pad(edge_index[1].astype(jnp.int32).reshape(NU, U),
                    ((0, UB), (0, 0)))
    srccat = jnp.stack([src2d, src2d + N])

    xcat = jnp.concatenate([x[:, :HC], x[:, HC:]], axis=0)

    cntA, cntB = _sc_count(dst2d)
    aggA, aggB = _sc_scatter(xcat, srccat, dst2d)
    h = _dense_layer(aggA, aggB, cntA, cntB, x,
                     W1l.T, W1r.T, b1l.reshape(1, D))
    hcat = jnp.concatenate([h[:, :HC], h[:, HC:]], axis=0)
    aggA2, aggB2 = _sc_scatter(hcat, srccat, dst2d)
    out = _dense_layer(aggA2, aggB2, cntA, cntB, h,
                       W2l.T, W2r.T, b2l.reshape(1, D))
    return out


# speedup vs baseline: 1.1680x; 1.1680x over previous
import functools

import jax
import jax.numpy as jnp
from jax import lax
from jax.experimental import pallas as pl
from jax.experimental.pallas import tpu as pltpu
from jax.experimental.pallas import tpu_sc as plsc

N = 10000
D = 128
E = 320000
HC = D // 2
U = 128
NU = E // U
UB = 8

_NC = 2
_NS = 16
_NW = _NC * _NS
NU_W = -(-NU // _NW)
CH = 40
NCH = N // CH

_SC_PARAMS = pltpu.CompilerParams(use_tc_tiling_on_sc=False)


def _zero_vmem(buf, rows, cols):
    def zrow(r, carry):
        for k in range(cols // 16):
            buf[r, pl.ds(k * 16, 16)] = jnp.zeros((16,), jnp.float32)
        return carry

    lax.fori_loop(0, rows, zrow, 0)


@functools.partial(
    pl.kernel,
    out_type=(
        jax.ShapeDtypeStruct((N, HC), jnp.float32),
        jax.ShapeDtypeStruct((N, HC), jnp.float32),
    ),
    mesh=plsc.VectorSubcoreMesh(core_axis_name="c", subcore_axis_name="s"),
    scratch_types=[
        pltpu.VMEM_SHARED((N, HC), jnp.float32),
        pltpu.VMEM((CH, HC), jnp.float32),
        pltpu.VMEM((UB, U), jnp.int32),
        pltpu.VMEM((UB, U), jnp.int32),
        pltpu.VMEM((UB, U), jnp.int32),
        pltpu.VMEM((UB, U), jnp.int32),
    ] + [pltpu.VMEM((U, HC), jnp.float32)] * UB + [
        pltpu.SemaphoreType.DMA((UB,)),
        pltpu.SemaphoreType.DMA((UB,)),
    ],
    compiler_params=_SC_PARAMS,
)
def _sc_scatter(xcat, srccat, dst2d, aggA, aggB,
                shared_agg, zbuf, src_b0, dst_b0, src_b1, dst_b1, *rest):
    rows = rest[:UB]
    gsem, ssem = rest[UB], rest[UB + 1]
    idx_sets = ((src_b0, dst_b0), (src_b1, dst_b1))
    c = lax.axis_index("c")
    s = lax.axis_index("s")

    _zero_vmem(zbuf, CH, HC)

    def ztile(t, carry):
        pltpu.sync_copy(zbuf, shared_agg.at[pl.ds(t * CH, CH)])
        return carry

    lax.fori_loop(NCH * s // _NS, NCH * (s + 1) // _NS, ztile, 0)
    plsc.subcore_barrier()

    u0 = NU * s // _NS
    u1 = NU * (s + 1) // _NS
    nbt = (u1 - u0 + UB - 1) // UB

    def wait_gather(j):
        pltpu.make_async_copy(xcat.at[pl.ds(0, U)], rows[j], gsem.at[j]).wait()

    def wait_scatter(j):
        pltpu.make_async_copy(rows[j], shared_agg.at[pl.ds(0, U)],
                              ssem.at[j]).wait()

    def one_batch(kb, base, src_b, dst_b, have_prev):
        pltpu.sync_copy(srccat.at[c, pl.ds(base, UB)], src_b)
        pltpu.sync_copy(dst2d.at[pl.ds(base, UB)], dst_b)

        for j in range(UB):
            @pl.when(base + j < u1)
            def _(j=j):
                if have_prev:
                    wait_scatter(j)
                else:
                    @pl.when(kb > 0)
                    def _():
                        wait_scatter(j)

                pltpu.async_copy(xcat.at[src_b.at[j]], rows[j], gsem.at[j])

        for j in range(UB):
            @pl.when(base + j < u1)
            def _(j=j):
                wait_gather(j)
                pltpu.async_copy(rows[j], shared_agg.at[dst_b.at[j]],
                                 ssem.at[j], add=True)

    def pair(kb, carry):
        base = u0 + 2 * kb * UB
        one_batch(kb, base, *idx_sets[0], have_prev=False)

        @pl.when(base + UB < u1)
        def _():
            one_batch(kb, base + UB, *idx_sets[1], have_prev=True)

        return carry

    lax.fori_loop(0, (nbt + 1) // 2, pair, 0)

    last_base = u0 + (nbt - 2) * UB
    for j in range(UB):
        @pl.when(last_base + j < u1)
        def _(j=j):
            wait_scatter(j)

    plsc.subcore_barrier()

    def wtile(t, carry):
        tile = pl.ds(t * CH, CH)

        @pl.when(c == 0)
        def _():
            pltpu.sync_copy(shared_agg.at[tile], aggA.at[tile])

        @pl.when(c == 1)
        def _():
            pltpu.sync_copy(shared_agg.at[tile], aggB.at[tile])

        return carry

    lax.fori_loop(NCH * s // _NS, NCH * (s + 1) // _NS, wtile, 0)


@functools.partial(
    pl.kernel,
    out_type=(
        jax.ShapeDtypeStruct((N, 16), jnp.float32),
        jax.ShapeDtypeStruct((N, 16), jnp.float32),
    ),
    mesh=plsc.VectorSubcoreMesh(core_axis_name="c", subcore_axis_name="s"),
    scratch_types=[
        pltpu.VMEM_SHARED((N, 16), jnp.float32),
        pltpu.VMEM((CH, 16), jnp.float32),
        pltpu.VMEM((U, 16), jnp.float32),
        pltpu.VMEM((NU_W, U), jnp.int32),
        pltpu.SemaphoreType.DMA,
    ],
    compiler_params=_SC_PARAMS,
)
def _sc_count(dst2d, cntA, cntB, shared_cnt, zbufc, ones_v, dst_all, csem):
    c = lax.axis_index("c")
    s = lax.axis_index("s")

    _zero_vmem(zbufc, CH, 16)

    def orow(r, carry):
        ones_v[r, pl.ds(0, 16)] = jnp.ones((16,), jnp.float32)
        return carry

    lax.fori_loop(0, U, orow, 0)

    def ztile(t, carry):
        pltpu.sync_copy(zbufc, shared_cnt.at[pl.ds(t * CH, CH)])
        return carry

    lax.fori_loop(NCH * s // _NS, NCH * (s + 1) // _NS, ztile, 0)
    plsc.subcore_barrier()

    w = s * _NC + c
    u0 = NU * w // _NW
    u1 = NU * (w + 1) // _NW
    pltpu.sync_copy(dst2d.at[pl.ds(u0, NU_W)], dst_all)

    def batch(k, carry):
        base = u0 + k * 8
        for b in range(8):
            @pl.when(base + b < u1)
            def _(u=base + b):
                pltpu.async_copy(ones_v, shared_cnt.at[dst_all.at[u - u0]],
                                 csem, add=True)
        for b in range(8):
            @pl.when(base + b < u1)
            def _():
                pltpu.make_async_copy(ones_v, shared_cnt.at[pl.ds(0, U)],
                                      csem).wait()
        return carry

    lax.fori_loop(0, (u1 - u0 + 7) // 8, batch, 0)
    plsc.subcore_barrier()

    def wtile(t, carry):
        tile = pl.ds(t * CH, CH)

        @pl.when(c == 0)
        def _():
            pltpu.sync_copy(shared_cnt.at[tile], cntA.at[tile])

        @pl.when(c == 1)
        def _():
            pltpu.sync_copy(shared_cnt.at[tile], cntB.at[tile])

        return carry

    lax.fori_loop(NCH * s // _NS, NCH * (s + 1) // _NS, wtile, 0)


_B = 512


def _dense_body(aA, aB, cA, cB, x, WlT, WrT, b, out):
    cnt = cA[:, 0:1] + cB[:, 0:1]
    inv = 1.0 / jnp.maximum(cnt, 1.0)
    m = jnp.concatenate([aA[...], aB[...]], axis=1) * inv
    z = (jnp.dot(m, WlT[...], preferred_element_type=jnp.float32)
         + jnp.dot(x[...], WrT[...], preferred_element_type=jnp.float32)
         + b[...])
    out[...] = jnp.maximum(z, 0.0)


def _dense_layer(aA, aB, cA, cB, x, WlT, WrT, b):
    grid = (pl.cdiv(N, _B),)
    row128 = pl.BlockSpec((_B, D), lambda i: (i, 0))
    row64 = pl.BlockSpec((_B, HC), lambda i: (i, 0))
    row16 = pl.BlockSpec((_B, 16), lambda i: (i, 0))
    wfull = pl.BlockSpec((D, D), lambda i: (0, 0))
    bfull = pl.BlockSpec((1, D), lambda i: (0, 0))
    return pl.pallas_call(
        _dense_body,
        grid=grid,
        in_specs=[row64, row64, row16, row16, row128, wfull, wfull, bfull],
        out_specs=row128,
        out_shape=jax.ShapeDtypeStruct((N, D), jnp.float32),
    )(aA, aB, cA, cB, x, WlT, WrT, b)


def kernel(x, edge_index, W1l, b1l, W1r, W2l, b2l, W2r):
    src2d = jnp.pad(edge_index[0].astype(jnp.int32).reshape(NU, U),
                    ((0, UB), (0, 0)))
    dst2d = jnp.pad(edge_index[1].astype(jnp.int32).reshape(NU, U),
                    ((0, UB), (0, 0)))
    srccat = jnp.stack([src2d, src2d + N])

    xcat = jnp.concatenate([x[:, :HC], x[:, HC:]], axis=0)

    cntA, cntB = _sc_count(dst2d)
    aggA, aggB = _sc_scatter(xcat, srccat, dst2d)
    h = _dense_layer(aggA, aggB, cntA, cntB, x,
                     W1l.T, W1r.T, b1l.reshape(1, D))
    hcat = jnp.concatenate([h[:, :HC], h[:, HC:]], axis=0)
    aggA2, aggB2 = _sc_scatter(hcat, srccat, dst2d)
    out = _dense_layer(aggA2, aggB2, cntA, cntB, h,
                       W2l.T, W2r.T, b2l.reshape(1, D))
    return out


# speedup vs baseline: 1.2526x; 1.0724x over previous
import functools

import jax
import jax.numpy as jnp
from jax import lax
from jax.experimental import pallas as pl
from jax.experimental.pallas import tpu as pltpu
from jax.experimental.pallas import tpu_sc as plsc

N = 10000
D = 128
E = 320000
U = 128
NU = E // U
NB = 2
UB = 8

_NC = 2
_NS = 16
_NW = _NC * _NS
NU_W = -(-NU // _NW)
CH = 25
NCH = N // CH

_SC_PARAMS = pltpu.CompilerParams(use_tc_tiling_on_sc=False)


def _zero_vmem(buf, rows, cols):
    def zrow(r, carry):
        for k in range(cols // 16):
            buf[r, pl.ds(k * 16, 16)] = jnp.zeros((16,), jnp.float32)
        return carry

    lax.fori_loop(0, rows, zrow, 0)


@functools.partial(
    pl.kernel,
    out_type=(
        jax.ShapeDtypeStruct((N, D), jnp.float32),
        jax.ShapeDtypeStruct((N, D), jnp.float32),
    ),
    mesh=plsc.VectorSubcoreMesh(core_axis_name="c", subcore_axis_name="s"),
    scratch_types=[
        pltpu.VMEM_SHARED((N, D), jnp.float32),
        pltpu.VMEM((CH, D), jnp.float32),
        pltpu.VMEM((UB, U), jnp.int32),
        pltpu.VMEM((UB, U), jnp.int32),
        pltpu.VMEM((UB, U), jnp.int32),
        pltpu.VMEM((UB, U), jnp.int32),
        pltpu.VMEM((U, D), jnp.float32),
        pltpu.VMEM((U, D), jnp.float32),
        pltpu.SemaphoreType.DMA((NB,)),
        pltpu.SemaphoreType.DMA((NB,)),
    ],
    compiler_params=_SC_PARAMS,
)
def _sc_scatter(x, src2d, dst2d, aggA, aggB,
                shared_agg, zbuf, src_b0, dst_b0, src_b1, dst_b1,
                rows0, rows1, gsem, ssem):
    rows = (rows0, rows1)
    idx_sets = ((src_b0, dst_b0), (src_b1, dst_b1))
    c = lax.axis_index("c")
    s = lax.axis_index("s")

    _zero_vmem(zbuf, CH, D)

    def ztile(t, carry):
        pltpu.sync_copy(zbuf, shared_agg.at[pl.ds(t * CH, CH)])
        return carry

    lax.fori_loop(NCH * s // _NS, NCH * (s + 1) // _NS, ztile, 0)
    plsc.subcore_barrier()

    w = s * _NC + c
    u0 = 2 * ((NU // 2) * w // _NW)
    u1 = 2 * ((NU // 2) * (w + 1) // _NW)
    nbt = (u1 - u0 + UB - 1) // UB

    def start_gather(src_b, j, b):
        pltpu.async_copy(x.at[src_b.at[j]], rows[b], gsem.at[b])

    def start_scatter(dst_b, j, b):
        pltpu.async_copy(rows[b], shared_agg.at[dst_b.at[j]],
                         ssem.at[b], add=True)

    def wait_gather(b):
        pltpu.make_async_copy(x.at[pl.ds(0, U)], rows[b], gsem.at[b]).wait()

    def wait_scatter(b):
        pltpu.make_async_copy(rows[b], shared_agg.at[pl.ds(0, U)],
                              ssem.at[b]).wait()

    def drain_prev(b, kb, have_prev):
        if have_prev:
            wait_scatter(b)
        else:
            @pl.when(kb > 0)
            def _():
                wait_scatter(b)

    def one_batch(kb, base, src_b, dst_b, have_prev):
        pltpu.sync_copy(src2d.at[pl.ds(base, UB)], src_b)
        pltpu.sync_copy(dst2d.at[pl.ds(base, UB)], dst_b)

        drain_prev(0, kb, have_prev)
        start_gather(src_b, 0, 0)

        for j in range(UB):
            b = j % NB

            @pl.when(base + j < u1)
            def _(j=j, b=b):
                wait_gather(b)
                start_scatter(dst_b, j, b)

            if j + 1 < UB:
                j2 = j + 1
                b2 = j2 % NB

                @pl.when(base + j2 < u1)
                def _(j2=j2, b2=b2):
                    if j2 >= NB:
                        wait_scatter(b2)
                    else:
                        drain_prev(b2, kb, have_prev)
                    start_gather(src_b, j2, b2)

    def pair(kb, carry):
        base = u0 + 2 * kb * UB
        one_batch(kb, base, *idx_sets[0], have_prev=False)

        @pl.when(base + UB < u1)
        def _():
            one_batch(kb, base + UB, *idx_sets[1], have_prev=True)

        return carry

    lax.fori_loop(0, (nbt + 1) // 2, pair, 0)

    @pl.when(u0 < u1)
    def _():
        wait_scatter(0)
        wait_scatter(1)

    plsc.subcore_barrier()

    def wtile(t, carry):
        tile = pl.ds(t * CH, CH)

        @pl.when(c == 0)
        def _():
            pltpu.sync_copy(shared_agg.at[tile], aggA.at[tile])

        @pl.when(c == 1)
        def _():
            pltpu.sync_copy(shared_agg.at[tile], aggB.at[tile])

        return carry

    lax.fori_loop(NCH * s // _NS, NCH * (s + 1) // _NS, wtile, 0)


@functools.partial(
    pl.kernel,
    out_type=(
        jax.ShapeDtypeStruct((N, 16), jnp.float32),
        jax.ShapeDtypeStruct((N, 16), jnp.float32),
    ),
    mesh=plsc.VectorSubcoreMesh(core_axis_name="c", subcore_axis_name="s"),
    scratch_types=[
        pltpu.VMEM_SHARED((N, 16), jnp.float32),
        pltpu.VMEM((CH, 16), jnp.float32),
        pltpu.VMEM((U, 16), jnp.float32),
        pltpu.VMEM((NU_W, U), jnp.int32),
        pltpu.SemaphoreType.DMA,
    ],
    compiler_params=_SC_PARAMS,
)
def _sc_count(dst2d, cntA, cntB, shared_cnt, zbufc, ones_v, dst_all, csem):
    c = lax.axis_index("c")
    s = lax.axis_index("s")

    _zero_vmem(zbufc, CH, 16)

    def orow(r, carry):
        ones_v[r, pl.ds(0, 16)] = jnp.ones((16,), jnp.float32)
        return carry

    lax.fori_loop(0, U, orow, 0)

    def ztile(t, carry):
        pltpu.sync_copy(zbufc, shared_cnt.at[pl.ds(t * CH, CH)])
        return carry

    lax.fori_loop(NCH * s // _NS, NCH * (s + 1) // _NS, ztile, 0)
    plsc.subcore_barrier()

    w = s * _NC + c
    u0 = NU * w // _NW
    u1 = NU * (w + 1) // _NW
    pltpu.sync_copy(dst2d.at[pl.ds(u0, NU_W)], dst_all)

    def batch(k, carry):
        base = u0 + k * 8
        for b in range(8):
            @pl.when(base + b < u1)
            def _(u=base + b):
                pltpu.async_copy(ones_v, shared_cnt.at[dst_all.at[u - u0]],
                                 csem, add=True)
        for b in range(8):
            @pl.when(base + b < u1)
            def _():
                pltpu.make_async_copy(ones_v, shared_cnt.at[pl.ds(0, U)],
                                      csem).wait()
        return carry

    lax.fori_loop(0, (u1 - u0 + 7) // 8, batch, 0)
    plsc.subcore_barrier()

    def wtile(t, carry):
        tile = pl.ds(t * CH, CH)

        @pl.when(c == 0)
        def _():
            pltpu.sync_copy(shared_cnt.at[tile], cntA.at[tile])

        @pl.when(c == 1)
        def _():
            pltpu.sync_copy(shared_cnt.at[tile], cntB.at[tile])

        return carry

    lax.fori_loop(NCH * s // _NS, NCH * (s + 1) // _NS, wtile, 0)


_B = 512


def _dense_body(aA, aB, cA, cB, x, WlT, WrT, b, out):
    cnt = cA[:, 0:1] + cB[:, 0:1]
    inv = 1.0 / jnp.maximum(cnt, 1.0)
    m = (aA[...] + aB[...]) * inv
    z = (jnp.dot(m, WlT[...], preferred_element_type=jnp.float32)
         + jnp.dot(x[...], WrT[...], preferred_element_type=jnp.float32)
         + b[...])
    out[...] = jnp.maximum(z, 0.0)


def _dense_layer(aA, aB, cA, cB, x, WlT, WrT, b):
    grid = (pl.cdiv(N, _B),)
    row128 = pl.BlockSpec((_B, D), lambda i: (i, 0))
    row16 = pl.BlockSpec((_B, 16), lambda i: (i, 0))
    wfull = pl.BlockSpec((D, D), lambda i: (0, 0))
    bfull = pl.BlockSpec((1, D), lambda i: (0, 0))
    return pl.pallas_call(
        _dense_body,
        grid=grid,
        in_specs=[row128, row128, row16, row16, row128, wfull, wfull, bfull],
        out_specs=row128,
        out_shape=jax.ShapeDtypeStruct((N, D), jnp.float32),
    )(aA, aB, cA, cB, x, WlT, WrT, b)


def kernel(x, edge_index, W1l, b1l, W1r, W2l, b2l, W2r):
    src2d = jnp.pad(edge_index[0].astype(jnp.int32).reshape(NU, U),
                    ((0, UB), (0, 0)))
    dst2d = jnp.pad(edge_index[1].astype(jnp.int32).reshape(NU, U),
                    ((0, UB), (0, 0)))

    cntA, cntB = _sc_count(dst2d)
    aggA, aggB = _sc_scatter(x, src2d, dst2d)
    h = _dense_layer(aggA, aggB, cntA, cntB, x,
                     W1l.T, W1r.T, b1l.reshape(1, D))
    aggA2, aggB2 = _sc_scatter(h, src2d, dst2d)
    out = _dense_layer(aggA2, aggB2, cntA, cntB, h,
                       W2l.T, W2r.T, b2l.reshape(1, D))
    return out


# speedup vs baseline: 1.2744x; 1.0174x over previous
import functools

import jax
import jax.numpy as jnp
from jax import lax
from jax.experimental import pallas as pl
from jax.experimental.pallas import tpu as pltpu
from jax.experimental.pallas import tpu_sc as plsc

N = 10000
D = 128
E = 320000
U = 128
NU = E // U
NB = 2
UB = 16

_NC = 2
_NS = 16
_NW = _NC * _NS
NU_W = -(-NU // _NW)
CH = 25
NCH = N // CH

_SC_PARAMS = pltpu.CompilerParams(use_tc_tiling_on_sc=False)


def _zero_vmem(buf, rows, cols):
    def zrow(r, carry):
        for k in range(cols // 16):
            buf[r, pl.ds(k * 16, 16)] = jnp.zeros((16,), jnp.float32)
        return carry

    lax.fori_loop(0, rows, zrow, 0)


@functools.partial(
    pl.kernel,
    out_type=(
        jax.ShapeDtypeStruct((N, D), jnp.float32),
        jax.ShapeDtypeStruct((N, D), jnp.float32),
    ),
    mesh=plsc.VectorSubcoreMesh(core_axis_name="c", subcore_axis_name="s"),
    scratch_types=[
        pltpu.VMEM_SHARED((N, D), jnp.float32),
        pltpu.VMEM((CH, D), jnp.float32),
        pltpu.VMEM((UB, U), jnp.int32),
        pltpu.VMEM((UB, U), jnp.int32),
        pltpu.VMEM((UB, U), jnp.int32),
        pltpu.VMEM((UB, U), jnp.int32),
        pltpu.VMEM((U, D), jnp.float32),
        pltpu.VMEM((U, D), jnp.float32),
        pltpu.SemaphoreType.DMA((NB,)),
        pltpu.SemaphoreType.DMA((NB,)),
    ],
    compiler_params=_SC_PARAMS,
)
def _sc_scatter(x, src2d, dst2d, aggA, aggB,
                shared_agg, zbuf, src_b0, dst_b0, src_b1, dst_b1,
                rows0, rows1, gsem, ssem):
    rows = (rows0, rows1)
    idx_sets = ((src_b0, dst_b0), (src_b1, dst_b1))
    c = lax.axis_index("c")
    s = lax.axis_index("s")

    _zero_vmem(zbuf, CH, D)

    def ztile(t, carry):
        pltpu.sync_copy(zbuf, shared_agg.at[pl.ds(t * CH, CH)])
        return carry

    lax.fori_loop(NCH * s // _NS, NCH * (s + 1) // _NS, ztile, 0)
    plsc.subcore_barrier()

    w = s * _NC + c
    u0 = 2 * ((NU // 2) * w // _NW)
    u1 = 2 * ((NU // 2) * (w + 1) // _NW)
    nbt = (u1 - u0 + UB - 1) // UB

    def start_gather(src_b, j, b):
        pltpu.async_copy(x.at[src_b.at[j]], rows[b], gsem.at[b])

    def start_scatter(dst_b, j, b):
        pltpu.async_copy(rows[b], shared_agg.at[dst_b.at[j]],
                         ssem.at[b], add=True)

    def wait_gather(b):
        pltpu.make_async_copy(x.at[pl.ds(0, U)], rows[b], gsem.at[b]).wait()

    def wait_scatter(b):
        pltpu.make_async_copy(rows[b], shared_agg.at[pl.ds(0, U)],
                              ssem.at[b]).wait()

    def drain_prev(b, kb, have_prev):
        if have_prev:
            wait_scatter(b)
        else:
            @pl.when(kb > 0)
            def _():
                wait_scatter(b)

    def one_batch(kb, base, src_b, dst_b, have_prev):
        pltpu.sync_copy(src2d.at[pl.ds(base, UB)], src_b)
        pltpu.sync_copy(dst2d.at[pl.ds(base, UB)], dst_b)

        drain_prev(0, kb, have_prev)
        start_gather(src_b, 0, 0)

        for j in range(UB):
            b = j % NB

            @pl.when(base + j < u1)
            def _(j=j, b=b):
                wait_gather(b)
                start_scatter(dst_b, j, b)

            if j + 1 < UB:
                j2 = j + 1
                b2 = j2 % NB

                @pl.when(base + j2 < u1)
                def _(j2=j2, b2=b2):
                    if j2 >= NB:
                        wait_scatter(b2)
                    else:
                        drain_prev(b2, kb, have_prev)
                    start_gather(src_b, j2, b2)

    def pair(kb, carry):
        base = u0 + 2 * kb * UB
        one_batch(kb, base, *idx_sets[0], have_prev=False)

        @pl.when(base + UB < u1)
        def _():
            one_batch(kb, base + UB, *idx_sets[1], have_prev=True)

        return carry

    lax.fori_loop(0, (nbt + 1) // 2, pair, 0)

    @pl.when(u0 < u1)
    def _():
        wait_scatter(0)
        wait_scatter(1)

    plsc.subcore_barrier()

    def wtile(t, carry):
        tile = pl.ds(t * CH, CH)

        @pl.when(c == 0)
        def _():
            pltpu.sync_copy(shared_agg.at[tile], aggA.at[tile])

        @pl.when(c == 1)
        def _():
            pltpu.sync_copy(shared_agg.at[tile], aggB.at[tile])

        return carry

    lax.fori_loop(NCH * s // _NS, NCH * (s + 1) // _NS, wtile, 0)


@functools.partial(
    pl.kernel,
    out_type=(
        jax.ShapeDtypeStruct((N, 16), jnp.float32),
        jax.ShapeDtypeStruct((N, 16), jnp.float32),
    ),
    mesh=plsc.VectorSubcoreMesh(core_axis_name="c", subcore_axis_name="s"),
    scratch_types=[
        pltpu.VMEM_SHARED((N, 16), jnp.float32),
        pltpu.VMEM((CH, 16), jnp.float32),
        pltpu.VMEM((U, 16), jnp.float32),
        pltpu.VMEM((NU_W, U), jnp.int32),
        pltpu.SemaphoreType.DMA,
    ],
    compiler_params=_SC_PARAMS,
)
def _sc_count(dst2d, cntA, cntB, shared_cnt, zbufc, ones_v, dst_all, csem):
    c = lax.axis_index("c")
    s = lax.axis_index("s")

    _zero_vmem(zbufc, CH, 16)

    def orow(r, carry):
        ones_v[r, pl.ds(0, 16)] = jnp.ones((16,), jnp.float32)
        return carry

    lax.fori_loop(0, U, orow, 0)

    def ztile(t, carry):
        pltpu.sync_copy(zbufc, shared_cnt.at[pl.ds(t * CH, CH)])
        return carry

    lax.fori_loop(NCH * s // _NS, NCH * (s + 1) // _NS, ztile, 0)
    plsc.subcore_barrier()

    w = s * _NC + c
    u0 = NU * w // _NW
    u1 = NU * (w + 1) // _NW
    pltpu.sync_copy(dst2d.at[pl.ds(u0, NU_W)], dst_all)

    def batch(k, carry):
        base = u0 + k * 8
        for b in range(8):
            @pl.when(base + b < u1)
            def _(u=base + b):
                pltpu.async_copy(ones_v, shared_cnt.at[dst_all.at[u - u0]],
                                 csem, add=True)
        for b in range(8):
            @pl.when(base + b < u1)
            def _():
                pltpu.make_async_copy(ones_v, shared_cnt.at[pl.ds(0, U)],
                                      csem).wait()
        return carry

    lax.fori_loop(0, (u1 - u0 + 7) // 8, batch, 0)
    plsc.subcore_barrier()

    def wtile(t, carry):
        tile = pl.ds(t * CH, CH)

        @pl.when(c == 0)
        def _():
            pltpu.sync_copy(shared_cnt.at[tile], cntA.at[tile])

        @pl.when(c == 1)
        def _():
            pltpu.sync_copy(shared_cnt.at[tile], cntB.at[tile])

        return carry

    lax.fori_loop(NCH * s // _NS, NCH * (s + 1) // _NS, wtile, 0)


_B = 512


def _dense_body(aA, aB, cA, cB, x, WlT, WrT, b, out):
    cnt = cA[:, 0:1] + cB[:, 0:1]
    inv = 1.0 / jnp.maximum(cnt, 1.0)
    m = (aA[...] + aB[...]) * inv
    z = (jnp.dot(m, WlT[...], preferred_element_type=jnp.float32)
         + jnp.dot(x[...], WrT[...], preferred_element_type=jnp.float32)
         + b[...])
    out[...] = jnp.maximum(z, 0.0)


def _dense_layer(aA, aB, cA, cB, x, WlT, WrT, b):
    grid = (pl.cdiv(N, _B),)
    row128 = pl.BlockSpec((_B, D), lambda i: (i, 0))
    row16 = pl.BlockSpec((_B, 16), lambda i: (i, 0))
    wfull = pl.BlockSpec((D, D), lambda i: (0, 0))
    bfull = pl.BlockSpec((1, D), lambda i: (0, 0))
    return pl.pallas_call(
        _dense_body,
        grid=grid,
        in_specs=[row128, row128, row16, row16, row128, wfull, wfull, bfull],
        out_specs=row128,
        out_shape=jax.ShapeDtypeStruct((N, D), jnp.float32),
    )(aA, aB, cA, cB, x, WlT, WrT, b)


def kernel(x, edge_index, W1l, b1l, W1r, W2l, b2l, W2r):
    src2d = jnp.pad(edge_index[0].astype(jnp.int32).reshape(NU, U),
                    ((0, UB), (0, 0)))
    dst2d = jnp.pad(edge_index[1].astype(jnp.int32).reshape(NU, U),
                    ((0, UB), (0, 0)))

    cntA, cntB = _sc_count(dst2d)
    aggA, aggB = _sc_scatter(x, src2d, dst2d)
    h = _dense_layer(aggA, aggB, cntA, cntB, x,
                     W1l.T, W1r.T, b1l.reshape(1, D))
    aggA2, aggB2 = _sc_scatter(h, src2d, dst2d)
    out = _dense_layer(aggA2, aggB2, cntA, cntB, h,
                       W2l.T, W2r.T, b2l.reshape(1, D))
    return out


# speedup vs baseline: 1.3268x; 1.0411x over previous
import functools

import jax
import jax.numpy as jnp
from jax import lax
from jax.experimental import pallas as pl
from jax.experimental.pallas import tpu as pltpu
from jax.experimental.pallas import tpu_sc as plsc

N = 10000
D = 128
E = 320000
U = 128
NU = E // U
NB = 2
UB = 16

_NC = 2
_NS = 16
_NW = _NC * _NS
NU_W = -(-NU // _NW)
CH = 25
NCH = N // CH

_SC_PARAMS = pltpu.CompilerParams(use_tc_tiling_on_sc=False)


def _zero_vmem(buf, rows, cols):
    def zrow(r, carry):
        for k in range(cols // 16):
            buf[r, pl.ds(k * 16, 16)] = jnp.zeros((16,), jnp.float32)
        return carry

    lax.fori_loop(0, rows, zrow, 0)


@functools.partial(
    pl.kernel,
    out_type=(
        jax.ShapeDtypeStruct((N, D), jnp.float32),
        jax.ShapeDtypeStruct((N, D), jnp.float32),
    ),
    mesh=plsc.VectorSubcoreMesh(core_axis_name="c", subcore_axis_name="s"),
    scratch_types=[
        pltpu.VMEM_SHARED((N, D), jnp.float32),
        pltpu.VMEM((CH, D), jnp.float32),
        pltpu.VMEM((UB, U), jnp.int32),
        pltpu.VMEM((UB, U), jnp.int32),
        pltpu.VMEM((UB, U), jnp.int32),
        pltpu.VMEM((UB, U), jnp.int32),
        pltpu.VMEM((U, D), jnp.float32),
        pltpu.VMEM((U, D), jnp.float32),
        pltpu.SemaphoreType.DMA((NB,)),
        pltpu.SemaphoreType.DMA((NB,)),
    ],
    compiler_params=_SC_PARAMS,
)
def _sc_scatter(x, src2d, dst2d, aggA, aggB,
                shared_agg, zbuf, src_b0, dst_b0, src_b1, dst_b1,
                rows0, rows1, gsem, ssem):
    rows = (rows0, rows1)
    idx_sets = ((src_b0, dst_b0), (src_b1, dst_b1))
    c = lax.axis_index("c")
    s = lax.axis_index("s")

    _zero_vmem(zbuf, CH, D)

    def ztile(t, carry):
        pltpu.sync_copy(zbuf, shared_agg.at[pl.ds(t * CH, CH)])
        return carry

    lax.fori_loop(NCH * s // _NS, NCH * (s + 1) // _NS, ztile, 0)
    plsc.subcore_barrier()

    w = s * _NC + c
    u0 = 2 * ((NU // 2) * w // _NW)
    u1 = 2 * ((NU // 2) * (w + 1) // _NW)
    nbt = (u1 - u0 + UB - 1) // UB

    def start_gather(src_b, j, b):
        pltpu.async_copy(x.at[src_b.at[j]], rows[b], gsem.at[b])

    def start_scatter(dst_b, j, b):
        pltpu.async_copy(rows[b], shared_agg.at[dst_b.at[j]],
                         ssem.at[b], add=True)

    def wait_gather(b):
        pltpu.make_async_copy(x.at[pl.ds(0, U)], rows[b], gsem.at[b]).wait()

    def wait_scatter(b):
        pltpu.make_async_copy(rows[b], shared_agg.at[pl.ds(0, U)],
                              ssem.at[b]).wait()

    def drain_prev(b, kb, have_prev):
        if have_prev:
            wait_scatter(b)
        else:
            @pl.when(kb > 0)
            def _():
                wait_scatter(b)

    def one_batch(kb, base, src_b, dst_b, have_prev):
        base_c = jnp.minimum(base, NU - UB)
        off = base - base_c
        pltpu.sync_copy(src2d.at[pl.ds(base_c, UB)], src_b)
        pltpu.sync_copy(dst2d.at[pl.ds(base_c, UB)], dst_b)

        drain_prev(0, kb, have_prev)
        start_gather(src_b, off, 0)

        for j in range(UB):
            b = j % NB

            @pl.when(base + j < u1)
            def _(j=j, b=b):
                wait_gather(b)
                start_scatter(dst_b, off + j, b)

            if j + 1 < UB:
                j2 = j + 1
                b2 = j2 % NB

                @pl.when(base + j2 < u1)
                def _(j2=j2, b2=b2):
                    if j2 >= NB:
                        wait_scatter(b2)
                    else:
                        drain_prev(b2, kb, have_prev)
                    start_gather(src_b, off + j2, b2)

    def pair(kb, carry):
        base = u0 + 2 * kb * UB
        one_batch(kb, base, *idx_sets[0], have_prev=False)

        @pl.when(base + UB < u1)
        def _():
            one_batch(kb, base + UB, *idx_sets[1], have_prev=True)

        return carry

    lax.fori_loop(0, (nbt + 1) // 2, pair, 0)

    @pl.when(u0 < u1)
    def _():
        wait_scatter(0)
        wait_scatter(1)

    plsc.subcore_barrier()

    def wtile(t, carry):
        tile = pl.ds(t * CH, CH)

        @pl.when(c == 0)
        def _():
            pltpu.sync_copy(shared_agg.at[tile], aggA.at[tile])

        @pl.when(c == 1)
        def _():
            pltpu.sync_copy(shared_agg.at[tile], aggB.at[tile])

        return carry

    lax.fori_loop(NCH * s // _NS, NCH * (s + 1) // _NS, wtile, 0)


@functools.partial(
    pl.kernel,
    out_type=(
        jax.ShapeDtypeStruct((N, 16), jnp.float32),
        jax.ShapeDtypeStruct((N, 16), jnp.float32),
    ),
    mesh=plsc.VectorSubcoreMesh(core_axis_name="c", subcore_axis_name="s"),
    scratch_types=[
        pltpu.VMEM_SHARED((N, 16), jnp.float32),
        pltpu.VMEM((CH, 16), jnp.float32),
        pltpu.VMEM((U, 16), jnp.float32),
        pltpu.VMEM((NU_W, U), jnp.int32),
        pltpu.SemaphoreType.DMA,
    ],
    compiler_params=_SC_PARAMS,
)
def _sc_count(dst2d, cntA, cntB, shared_cnt, zbufc, ones_v, dst_all, csem):
    c = lax.axis_index("c")
    s = lax.axis_index("s")

    _zero_vmem(zbufc, CH, 16)

    def orow(r, carry):
        ones_v[r, pl.ds(0, 16)] = jnp.ones((16,), jnp.float32)
        return carry

    lax.fori_loop(0, U, orow, 0)

    def ztile(t, carry):
        pltpu.sync_copy(zbufc, shared_cnt.at[pl.ds(t * CH, CH)])
        return carry

    lax.fori_loop(NCH * s // _NS, NCH * (s + 1) // _NS, ztile, 0)
    plsc.subcore_barrier()

    w = s * _NC + c
    u0 = NU * w // _NW
    u1 = NU * (w + 1) // _NW
    pltpu.sync_copy(dst2d.at[pl.ds(u0, NU_W)], dst_all)

    def batch(k, carry):
        base = u0 + k * 8
        for b in range(8):
            @pl.when(base + b < u1)
            def _(u=base + b):
                pltpu.async_copy(ones_v, shared_cnt.at[dst_all.at[u - u0]],
                                 csem, add=True)
        for b in range(8):
            @pl.when(base + b < u1)
            def _():
                pltpu.make_async_copy(ones_v, shared_cnt.at[pl.ds(0, U)],
                                      csem).wait()
        return carry

    lax.fori_loop(0, (u1 - u0 + 7) // 8, batch, 0)
    plsc.subcore_barrier()

    def wtile(t, carry):
        tile = pl.ds(t * CH, CH)

        @pl.when(c == 0)
        def _():
            pltpu.sync_copy(shared_cnt.at[tile], cntA.at[tile])

        @pl.when(c == 1)
        def _():
            pltpu.sync_copy(shared_cnt.at[tile], cntB.at[tile])

        return carry

    lax.fori_loop(NCH * s // _NS, NCH * (s + 1) // _NS, wtile, 0)


_B = 1280


def _dense_body(aA, aB, cA, cB, x, WlT, WrT, b, out):
    cnt = cA[:, 0:1] + cB[:, 0:1]
    inv = 1.0 / jnp.maximum(cnt, 1.0)
    m = (aA[...] + aB[...]) * inv
    z = (jnp.dot(m, WlT[...], preferred_element_type=jnp.float32)
         + jnp.dot(x[...], WrT[...], preferred_element_type=jnp.float32)
         + b[...])
    out[...] = jnp.maximum(z, 0.0)


def _dense_layer(aA, aB, cA, cB, x, WlT, WrT, b):
    grid = (pl.cdiv(N, _B),)
    row128 = pl.BlockSpec((_B, D), lambda i: (i, 0))
    row16 = pl.BlockSpec((_B, 16), lambda i: (i, 0))
    wfull = pl.BlockSpec((D, D), lambda i: (0, 0))
    bfull = pl.BlockSpec((1, D), lambda i: (0, 0))
    return pl.pallas_call(
        _dense_body,
        grid=grid,
        in_specs=[row128, row128, row16, row16, row128, wfull, wfull, bfull],
        out_specs=row128,
        out_shape=jax.ShapeDtypeStruct((N, D), jnp.float32),
    )(aA, aB, cA, cB, x, WlT, WrT, b)


def kernel(x, edge_index, W1l, b1l, W1r, W2l, b2l, W2r):
    src2d = edge_index[0].astype(jnp.int32).reshape(NU, U)
    dst2d = edge_index[1].astype(jnp.int32).reshape(NU, U)

    cntA, cntB = _sc_count(dst2d)
    aggA, aggB = _sc_scatter(x, src2d, dst2d)
    h = _dense_layer(aggA, aggB, cntA, cntB, x,
                     W1l.T, W1r.T, b1l.reshape(1, D))
    aggA2, aggB2 = _sc_scatter(h, src2d, dst2d)
    out = _dense_layer(aggA2, aggB2, cntA, cntB, h,
                       W2l.T, W2r.T, b2l.reshape(1, D))
    return out


# speedup vs baseline: 1.3698x; 1.0324x over previous
import functools

import jax
import jax.numpy as jnp
from jax import lax
from jax.experimental import pallas as pl
from jax.experimental.pallas import tpu as pltpu
from jax.experimental.pallas import tpu_sc as plsc

N = 10000
D = 128
E = 320000
U = 128
NU = E // U
NB = 2
UB = 16

_NC = 2
_NS = 16
_NW = _NC * _NS
NU_W = -(-NU // _NW)
CH = 25
NCH = N // CH

_SC_PARAMS = pltpu.CompilerParams(use_tc_tiling_on_sc=False)


def _zero_vmem(buf, rows, cols):
    def zrow(r, carry):
        for k in range(cols // 16):
            buf[r, pl.ds(k * 16, 16)] = jnp.zeros((16,), jnp.float32)
        return carry

    lax.fori_loop(0, rows, zrow, 0)


@functools.partial(
    pl.kernel,
    out_type=(
        jax.ShapeDtypeStruct((N, D), jnp.float32),
        jax.ShapeDtypeStruct((N, D), jnp.float32),
    ),
    mesh=plsc.VectorSubcoreMesh(core_axis_name="c", subcore_axis_name="s"),
    scratch_types=[
        pltpu.VMEM_SHARED((N, D), jnp.float32),
        pltpu.VMEM((CH, D), jnp.float32),
        pltpu.VMEM((UB, U), jnp.int32),
        pltpu.VMEM((UB, U), jnp.int32),
        pltpu.VMEM((UB, U), jnp.int32),
        pltpu.VMEM((UB, U), jnp.int32),
        pltpu.VMEM((U, D), jnp.float32),
        pltpu.VMEM((U, D), jnp.float32),
        pltpu.SemaphoreType.DMA((NB,)),
        pltpu.SemaphoreType.DMA((NB,)),
    ],
    compiler_params=_SC_PARAMS,
)
def _sc_scatter(x, ei3, aggA, aggB,
                shared_agg, zbuf, src_b0, dst_b0, src_b1, dst_b1,
                rows0, rows1, gsem, ssem):
    rows = (rows0, rows1)
    idx_sets = ((src_b0, dst_b0), (src_b1, dst_b1))
    c = lax.axis_index("c")
    s = lax.axis_index("s")

    _zero_vmem(zbuf, CH, D)

    def ztile(t, carry):
        pltpu.sync_copy(zbuf, shared_agg.at[pl.ds(t * CH, CH)])
        return carry

    lax.fori_loop(NCH * s // _NS, NCH * (s + 1) // _NS, ztile, 0)
    plsc.subcore_barrier()

    w = s * _NC + c
    u0 = 2 * ((NU // 2) * w // _NW)
    u1 = 2 * ((NU // 2) * (w + 1) // _NW)
    nbt = (u1 - u0 + UB - 1) // UB

    def start_gather(src_b, j, b):
        pltpu.async_copy(x.at[src_b.at[j]], rows[b], gsem.at[b])

    def start_scatter(dst_b, j, b):
        pltpu.async_copy(rows[b], shared_agg.at[dst_b.at[j]],
                         ssem.at[b], add=True)

    def wait_gather(b):
        pltpu.make_async_copy(x.at[pl.ds(0, U)], rows[b], gsem.at[b]).wait()

    def wait_scatter(b):
        pltpu.make_async_copy(rows[b], shared_agg.at[pl.ds(0, U)],
                              ssem.at[b]).wait()

    def drain_prev(b, kb, have_prev):
        if have_prev:
            wait_scatter(b)
        else:
            @pl.when(kb > 0)
            def _():
                wait_scatter(b)

    def one_batch(kb, base, src_b, dst_b, have_prev):
        base_c = jnp.minimum(base, NU - UB)
        off = base - base_c
        pltpu.sync_copy(ei3.at[0, pl.ds(base_c, UB)], src_b)
        pltpu.sync_copy(ei3.at[1, pl.ds(base_c, UB)], dst_b)

        drain_prev(0, kb, have_prev)
        start_gather(src_b, off, 0)

        for j in range(UB):
            b = j % NB

            @pl.when(base + j < u1)
            def _(j=j, b=b):
                wait_gather(b)
                start_scatter(dst_b, off + j, b)

            if j + 1 < UB:
                j2 = j + 1
                b2 = j2 % NB

                @pl.when(base + j2 < u1)
                def _(j2=j2, b2=b2):
                    if j2 >= NB:
                        wait_scatter(b2)
                    else:
                        drain_prev(b2, kb, have_prev)
                    start_gather(src_b, off + j2, b2)

    def pair(kb, carry):
        base = u0 + 2 * kb * UB
        one_batch(kb, base, *idx_sets[0], have_prev=False)

        @pl.when(base + UB < u1)
        def _():
            one_batch(kb, base + UB, *idx_sets[1], have_prev=True)

        return carry

    lax.fori_loop(0, (nbt + 1) // 2, pair, 0)

    @pl.when(u0 < u1)
    def _():
        wait_scatter(0)
        wait_scatter(1)

    plsc.subcore_barrier()

    def wtile(t, carry):
        tile = pl.ds(t * CH, CH)

        @pl.when(c == 0)
        def _():
            pltpu.sync_copy(shared_agg.at[tile], aggA.at[tile])

        @pl.when(c == 1)
        def _():
            pltpu.sync_copy(shared_agg.at[tile], aggB.at[tile])

        return carry

    lax.fori_loop(NCH * s // _NS, NCH * (s + 1) // _NS, wtile, 0)


@functools.partial(
    pl.kernel,
    out_type=(
        jax.ShapeDtypeStruct((N, 16), jnp.float32),
        jax.ShapeDtypeStruct((N, 16), jnp.float32),
    ),
    mesh=plsc.VectorSubcoreMesh(core_axis_name="c", subcore_axis_name="s"),
    scratch_types=[
        pltpu.VMEM_SHARED((N, 16), jnp.float32),
        pltpu.VMEM((CH, 16), jnp.float32),
        pltpu.VMEM((U, 16), jnp.float32),
        pltpu.VMEM((NU_W, U), jnp.int32),
        pltpu.SemaphoreType.DMA,
    ],
    compiler_params=_SC_PARAMS,
)
def _sc_count(ei3, cntA, cntB, shared_cnt, zbufc, ones_v, dst_all, csem):
    c = lax.axis_index("c")
    s = lax.axis_index("s")

    _zero_vmem(zbufc, CH, 16)

    def orow(r, carry):
        ones_v[r, pl.ds(0, 16)] = jnp.ones((16,), jnp.float32)
        return carry

    lax.fori_loop(0, U, orow, 0)

    def ztile(t, carry):
        pltpu.sync_copy(zbufc, shared_cnt.at[pl.ds(t * CH, CH)])
        return carry

    lax.fori_loop(NCH * s // _NS, NCH * (s + 1) // _NS, ztile, 0)
    plsc.subcore_barrier()

    w = s * _NC + c
    u0 = NU * w // _NW
    u1 = NU * (w + 1) // _NW
    pltpu.sync_copy(ei3.at[1, pl.ds(u0, NU_W)], dst_all)

    def batch(k, carry):
        base = u0 + k * 8
        for b in range(8):
            @pl.when(base + b < u1)
            def _(u=base + b):
                pltpu.async_copy(ones_v, shared_cnt.at[dst_all.at[u - u0]],
                                 csem, add=True)
        for b in range(8):
            @pl.when(base + b < u1)
            def _():
                pltpu.make_async_copy(ones_v, shared_cnt.at[pl.ds(0, U)],
                                      csem).wait()
        return carry

    lax.fori_loop(0, (u1 - u0 + 7) // 8, batch, 0)
    plsc.subcore_barrier()

    def wtile(t, carry):
        tile = pl.ds(t * CH, CH)

        @pl.when(c == 0)
        def _():
            pltpu.sync_copy(shared_cnt.at[tile], cntA.at[tile])

        @pl.when(c == 1)
        def _():
            pltpu.sync_copy(shared_cnt.at[tile], cntB.at[tile])

        return carry

    lax.fori_loop(NCH * s // _NS, NCH * (s + 1) // _NS, wtile, 0)


_B = 1280


def _dense_body(aA, aB, cnt, x, WlT, WrT, b, out):
    inv = 1.0 / jnp.maximum(cnt[...], 1.0)
    m = (aA[...] + aB[...]) * inv
    z = (jnp.dot(m, WlT[...], preferred_element_type=jnp.float32)
         + jnp.dot(x[...], WrT[...], preferred_element_type=jnp.float32)
         + b[...])
    out[...] = jnp.maximum(z, 0.0)


def _dense_layer(aA, aB, cnt_col, x, WlT, WrT, b):
    grid = (pl.cdiv(N, _B),)
    row128 = pl.BlockSpec((_B, D), lambda i: (i, 0))
    row1 = pl.BlockSpec((_B, 1), lambda i: (i, 0))
    wfull = pl.BlockSpec((D, D), lambda i: (0, 0))
    bfull = pl.BlockSpec((1, D), lambda i: (0, 0))
    return pl.pallas_call(
        _dense_body,
        grid=grid,
        in_specs=[row128, row128, row1, row128, wfull, wfull, bfull],
        out_specs=row128,
        out_shape=jax.ShapeDtypeStruct((N, D), jnp.float32),
    )(aA, aB, cnt_col, x, WlT, WrT, b)


def kernel(x, edge_index, W1l, b1l, W1r, W2l, b2l, W2r):
    ei3 = edge_index.astype(jnp.int32).reshape(2, NU, U)

    cntA, cntB = _sc_count(ei3)
    aggA, aggB = _sc_scatter(x, ei3)
    cnt_col = cntA[:, :1] + cntB[:, :1]
    h = _dense_layer(aggA, aggB, cnt_col, x,
                     W1l.T, W1r.T, b1l.reshape(1, D))
    aggA2, aggB2 = _sc_scatter(h, ei3)
    out = _dense_layer(aggA2, aggB2, cnt_col, h,
                       W2l.T, W2r.T, b2l.reshape(1, D))
    return out
